# baseline (speedup 1.0000x reference)
import jax
import jax.numpy as jnp
from jax.experimental import pallas as pl
from jax.experimental.pallas import tpu as pltpu

_L, _N, _V, _D, _HID = 20, 1024, 7, 16, 256
_T = _L * _N
_C = _V * _V * 3
_CP = 152
_W = 8 * _CP + 16
_TB = 2048


def _proj_kernel(tsel_ref, w1_ref, at_ref, p_ref):
    w1 = w1_ref[...].reshape(_C + 1, _D, _HID)
    zero5 = jnp.zeros((_CP - _C, _HID), jnp.float32)
    w1ds = [jnp.concatenate([w1[:_C, d, :], zero5], axis=0)
            for d in range(_D)]
    for v in range(8):
        acc = tsel_ref[_CP * v:_CP * (v + 1), 0:1] * w1ds[0]
        for d in range(1, _D):
            acc = acc + tsel_ref[_CP * v:_CP * (v + 1), d:d + 1] * w1ds[d]
        p_ref[_CP * v:_CP * (v + 1), :] = acc.astype(jnp.bfloat16)
    acta = at_ref[:, 0:1] * w1[_C, 0, :][None, :]
    for d in range(1, _D):
        acta = acta + at_ref[:, d:d + 1] * w1[_C, d, :][None, :]
    p_ref[8 * _CP:, :] = acta.astype(jnp.bfloat16)


def _mlp_kernel(img_ref, act_ref, p_ref, w2_ref, w3_ref, b1_ref, b2_ref,
                b3_ref, out_ref):
    img = img_ref[...]
    h1 = None
    for v in range(8):
        eq = (img == v).astype(jnp.bfloat16)
        d = jnp.dot(eq, p_ref[_CP * v:_CP * v + _C, :],
                    preferred_element_type=jnp.float32)
        h1 = d if h1 is None else h1 + d
    oh_act = (act_ref[...] == jax.lax.broadcasted_iota(
        jnp.int32, (_TB, 16), 1)).astype(jnp.bfloat16)
    h1 += jnp.dot(oh_act, p_ref[8 * _CP:, :], preferred_element_type=jnp.float32)
    h1 = jnp.tanh(h1 + b1_ref[...]).astype(jnp.bfloat16)
    h2 = jnp.dot(h1, w2_ref[...].astype(jnp.bfloat16),
                 preferred_element_type=jnp.float32)
    h2 = jnp.tanh(h2 + b2_ref[...])
    out_ref[...] = jnp.sum(h2 * w3_ref[...], axis=1, keepdims=True) + b3_ref[...]


def _build_tsel(ot, ct, st):
    tcat = jnp.concatenate([ot[:8], ct[:8], st[:8], jnp.zeros((1, _D), ot.dtype)],
                           axis=0)
    rows = jnp.arange(8 * _CP)
    v, c = rows // _CP, rows % _CP
    idx = jnp.where(c < _C, 8 * (c % 3) + v, 24)
    return jnp.take(tcat, idx, axis=0)


def kernel(minigrid_ego_image, actions, object_table, color_table, state_table,
           action_table, W1, b1, W2, b2, W3, b3):
    img = minigrid_ego_image.reshape(_T, _C).astype(jnp.float32)
    act = actions.reshape(_T, 1)

    tsel = _build_tsel(object_table, color_table, state_table)
    p_tab = pl.pallas_call(
        _proj_kernel,
        out_shape=jax.ShapeDtypeStruct((_W, _HID), jnp.bfloat16),
    )(tsel, W1, action_table)

    grid = (_T // _TB,)
    out = pl.pallas_call(
        _mlp_kernel,
        grid=grid,
        in_specs=[
            pl.BlockSpec((_TB, _C), lambda i: (i, 0)),
            pl.BlockSpec((_TB, 1), lambda i: (i, 0)),
            pl.BlockSpec((_W, _HID), lambda i: (0, 0)),
            pl.BlockSpec((_HID, _HID), lambda i: (0, 0)),
            pl.BlockSpec((1, _HID), lambda i: (0, 0)),
            pl.BlockSpec((1, _HID), lambda i: (0, 0)),
            pl.BlockSpec((1, _HID), lambda i: (0, 0)),
            pl.BlockSpec((1, 1), lambda i: (0, 0)),
        ],
        out_specs=pl.BlockSpec((_TB, 1), lambda i: (i, 0)),
        out_shape=jax.ShapeDtypeStruct((_T, 1), jnp.float32),
        compiler_params=pltpu.CompilerParams(
            dimension_semantics=("parallel",)),
    )(img, act, p_tab, W2, W3.reshape(1, _HID),
      b1.reshape(1, _HID), b2.reshape(1, _HID), b3.reshape(1, 1))

    return out.reshape(_L, _N, 1)


# speedup vs baseline: 1.6202x; 1.6202x over previous
import jax
import jax.numpy as jnp
from jax.experimental import pallas as pl
from jax.experimental.pallas import tpu as pltpu

_L, _N, _V, _D, _HID = 20, 1024, 7, 16, 256
_T = _L * _N
_C = _V * _V * 3
_CP = 152
_W = 8 * _CP + 16
_TB = 2048


def _proj_kernel(tsel_ref, w1_ref, at_ref, p_ref):
    w1 = w1_ref[...].reshape(_C + 1, _D, _HID)
    zero5 = jnp.zeros((_CP - _C, _HID), jnp.float32)
    w1ds = [jnp.concatenate([w1[:_C, d, :], zero5], axis=0)
            for d in range(_D)]
    for v in range(8):
        acc = tsel_ref[_CP * v:_CP * (v + 1), 0:1] * w1ds[0]
        for d in range(1, _D):
            acc = acc + tsel_ref[_CP * v:_CP * (v + 1), d:d + 1] * w1ds[d]
        p_ref[_CP * v:_CP * (v + 1), :] = acc.astype(jnp.bfloat16)
    acta = at_ref[:, 0:1] * w1[_C, 0, :][None, :]
    for d in range(1, _D):
        acta = acta + at_ref[:, d:d + 1] * w1[_C, d, :][None, :]
    p_ref[8 * _CP:, :] = acta.astype(jnp.bfloat16)


def _mlp_kernel(img_ref, act_ref, p_ref, w2_ref, w3_ref, b1_ref, b2_ref,
                b3_ref, out_ref):
    img = img_ref[...].astype(jnp.float32)
    h1 = None
    for v in range(8):
        eq = (img == v).astype(jnp.bfloat16)
        d = jnp.dot(eq, p_ref[_CP * v:_CP * v + _C, :],
                    preferred_element_type=jnp.float32)
        h1 = d if h1 is None else h1 + d
    oh_act = (act_ref[...] == jax.lax.broadcasted_iota(
        jnp.int32, (_TB, 16), 1)).astype(jnp.bfloat16)
    h1 += jnp.dot(oh_act, p_ref[8 * _CP:, :], preferred_element_type=jnp.float32)
    h1 = jnp.tanh(h1 + b1_ref[...]).astype(jnp.bfloat16)
    h2 = jnp.dot(h1, w2_ref[...].astype(jnp.bfloat16),
                 preferred_element_type=jnp.float32)
    h2 = jnp.tanh(h2 + b2_ref[...])
    out_ref[...] = jnp.sum(h2 * w3_ref[...], axis=1, keepdims=True) + b3_ref[...]


def _build_tsel(ot, ct, st):
    tcat = jnp.concatenate([ot[:8], ct[:8], st[:8], jnp.zeros((1, _D), ot.dtype)],
                           axis=0)
    rows = jnp.arange(8 * _CP)
    v, c = rows // _CP, rows % _CP
    idx = jnp.where(c < _C, 8 * (c % 3) + v, 24)
    return jnp.take(tcat, idx, axis=0)


def kernel(minigrid_ego_image, actions, object_table, color_table, state_table,
           action_table, W1, b1, W2, b2, W3, b3):
    img = minigrid_ego_image.reshape(_T, _C).astype(jnp.int8)
    act = actions.reshape(_T, 1)

    tsel = _build_tsel(object_table, color_table, state_table)
    p_tab = pl.pallas_call(
        _proj_kernel,
        out_shape=jax.ShapeDtypeStruct((_W, _HID), jnp.bfloat16),
    )(tsel, W1, action_table)

    grid = (_T // _TB,)
    out = pl.pallas_call(
        _mlp_kernel,
        grid=grid,
        in_specs=[
            pl.BlockSpec((_TB, _C), lambda i: (i, 0)),
            pl.BlockSpec((_TB, 1), lambda i: (i, 0)),
            pl.BlockSpec((_W, _HID), lambda i: (0, 0)),
            pl.BlockSpec((_HID, _HID), lambda i: (0, 0)),
            pl.BlockSpec((1, _HID), lambda i: (0, 0)),
            pl.BlockSpec((1, _HID), lambda i: (0, 0)),
            pl.BlockSpec((1, _HID), lambda i: (0, 0)),
            pl.BlockSpec((1, 1), lambda i: (0, 0)),
        ],
        out_specs=pl.BlockSpec((_TB, 1), lambda i: (i, 0)),
        out_shape=jax.ShapeDtypeStruct((_T, 1), jnp.float32),
        compiler_params=pltpu.CompilerParams(
            dimension_semantics=("parallel",)),
    )(img, act, p_tab, W2, W3.reshape(1, _HID),
      b1.reshape(1, _HID), b2.reshape(1, _HID), b3.reshape(1, 1))

    return out.reshape(_L, _N, 1)


# speedup vs baseline: 1.6569x; 1.0227x over previous
import jax
import jax.numpy as jnp
from jax.experimental import pallas as pl
from jax.experimental.pallas import tpu as pltpu

_L, _N, _V, _D, _HID = 20, 1024, 7, 16, 256
_T = _L * _N
_C = _V * _V * 3
_CP = 152
_W = 8 * _CP + 16
_TB = 2048


def _proj_kernel(tsel_ref, w1_ref, at_ref, p_ref):
    w1 = w1_ref[...].reshape(_C + 1, _D, _HID)
    zero5 = jnp.zeros((_CP - _C, _HID), jnp.float32)
    w1ds = [jnp.concatenate([w1[:_C, d, :], zero5], axis=0)
            for d in range(_D)]
    for v in range(8):
        acc = tsel_ref[_CP * v:_CP * (v + 1), 0:1] * w1ds[0]
        for d in range(1, _D):
            acc = acc + tsel_ref[_CP * v:_CP * (v + 1), d:d + 1] * w1ds[d]
        p_ref[_CP * v:_CP * (v + 1), :] = acc.astype(jnp.bfloat16)
    acta = at_ref[:, 0:1] * w1[_C, 0, :][None, :]
    for d in range(1, _D):
        acta = acta + at_ref[:, d:d + 1] * w1[_C, d, :][None, :]
    p_ref[8 * _CP:, :] = acta.astype(jnp.bfloat16)


def _mlp_kernel(img_ref, act_ref, p_ref, w2_ref, w3_ref, b1_ref, b2_ref,
                b3_ref, out_ref):
    img = img_ref[...].astype(jnp.float32)
    h1 = None
    for v in range(8):
        eq = (img == v).astype(jnp.bfloat16)
        d = jnp.dot(eq, p_ref[_CP * v:_CP * v + _C, :],
                    preferred_element_type=jnp.float32)
        h1 = d if h1 is None else h1 + d
    oh_act = (act_ref[...].astype(jnp.float32) == jax.lax.broadcasted_iota(
        jnp.int32, (_TB, 16), 1).astype(jnp.float32)).astype(jnp.bfloat16)
    h1 += jnp.dot(oh_act, p_ref[8 * _CP:, :], preferred_element_type=jnp.float32)
    h1 = jnp.tanh(h1 + b1_ref[...]).astype(jnp.bfloat16)
    h2 = jnp.dot(h1, w2_ref[...].astype(jnp.bfloat16),
                 preferred_element_type=jnp.float32)
    h2 = jnp.tanh(h2 + b2_ref[...])
    out_ref[...] = jnp.sum(h2 * w3_ref[...], axis=1, keepdims=True) + b3_ref[...]


def _build_tsel(ot, ct, st):
    tcat = jnp.concatenate([ot[:8], ct[:8], st[:8], jnp.zeros((1, _D), ot.dtype)],
                           axis=0)
    rows = jnp.arange(8 * _CP)
    v, c = rows // _CP, rows % _CP
    idx = jnp.where(c < _C, 8 * (c % 3) + v, 24)
    return jnp.take(tcat, idx, axis=0)


def kernel(minigrid_ego_image, actions, object_table, color_table, state_table,
           action_table, W1, b1, W2, b2, W3, b3):
    img = minigrid_ego_image.reshape(_T, _C).astype(jnp.int8)
    act = actions.reshape(_T, 1).astype(jnp.int8)

    tsel = _build_tsel(object_table, color_table, state_table)
    p_tab = pl.pallas_call(
        _proj_kernel,
        out_shape=jax.ShapeDtypeStruct((_W, _HID), jnp.bfloat16),
    )(tsel, W1, action_table)

    grid = (_T // _TB,)
    out = pl.pallas_call(
        _mlp_kernel,
        grid=grid,
        in_specs=[
            pl.BlockSpec((_TB, _C), lambda i: (i, 0)),
            pl.BlockSpec((_TB, 1), lambda i: (i, 0)),
            pl.BlockSpec((_W, _HID), lambda i: (0, 0)),
            pl.BlockSpec((_HID, _HID), lambda i: (0, 0)),
            pl.BlockSpec((1, _HID), lambda i: (0, 0)),
            pl.BlockSpec((1, _HID), lambda i: (0, 0)),
            pl.BlockSpec((1, _HID), lambda i: (0, 0)),
            pl.BlockSpec((1, 1), lambda i: (0, 0)),
        ],
        out_specs=pl.BlockSpec((_TB, 1), lambda i: (i, 0)),
        out_shape=jax.ShapeDtypeStruct((_T, 1), jnp.float32),
        compiler_params=pltpu.CompilerParams(
            dimension_semantics=("parallel",)),
    )(img, act, p_tab, W2, W3.reshape(1, _HID),
      b1.reshape(1, _HID), b2.reshape(1, _HID), b3.reshape(1, 1))

    return out.reshape(_L, _N, 1)
